# baseline (speedup 1.0000x reference)
import functools

import jax
import jax.numpy as jnp
from jax import lax
from jax.experimental import pallas as pl
from jax.experimental.pallas import tpu as pltpu
from jax.experimental.pallas import tpu_sc as plsc

N_U = 10000
N_I = 10000
E_TOT = 160000
D = 256
DH = 128
N_TILES = 16
CHUNK = 80
GCH = 25
NGROUP = 5
EDGES_PER_TILE = CHUNK * GCH * NGROUP
N_PAD = 10240
ROWS_PER_TILE = N_PAD // N_TILES
RB = 1000


def _spmm_body(dst0_hbm, dst1_hbm, src_hbm, ev_hbm, item_hbm, out_hbm,
               dst_v, src_v, ev_v, rows0, rows1, acc, gsem0, gsem1):
    c = lax.axis_index("c")
    s = lax.axis_index("s")
    rbase = s * ROWS_PER_TILE

    zero16 = jnp.zeros((16,), jnp.float32)

    def _zb(i, carry):
        for j in range(DH // 16):
            rows0[i, pl.ds(j * 16, 16)] = zero16
        return carry

    lax.fori_loop(0, CHUNK, _zb, 0)
    for r in range(ROWS_PER_TILE // CHUNK):
        pltpu.sync_copy(rows0, acc.at[pl.ds(rbase + r * CHUNK, CHUNK)])
    plsc.subcore_barrier()

    def _scale(buf, j):
        for gg in range(CHUNK // 16):
            evv = ev_v[j, pl.ds(gg * 16, 16)]
            for lane in range(16):
                val = evv[lane]
                e = gg * 16 + lane
                for k in range(DH // 16):
                    sl = pl.ds(k * 16, 16)
                    buf[e, sl] = buf[e, sl] * val

    def _gstart(j, buf, sem):
        pltpu.make_async_copy(item_hbm.at[dst_v.at[j]], buf, sem).start()

    def _gwait(j, buf, sem):
        pltpu.make_async_copy(item_hbm.at[dst_v.at[j]], buf, sem).wait()

    for g in range(NGROUP):
        @pl.when(c == 0)
        def _():
            pltpu.sync_copy(dst0_hbm.at[s, g], dst_v)

        @pl.when(c == 1)
        def _():
            pltpu.sync_copy(dst1_hbm.at[s, g], dst_v)

        pltpu.sync_copy(src_hbm.at[s, g], src_v)
        pltpu.sync_copy(ev_hbm.at[s, g], ev_v)

        _gstart(0, rows0, gsem0)

        def _pair(i, carry):
            j0 = i * 2
            _gwait(j0, rows0, gsem0)

            @pl.when(j0 + 1 < GCH)
            def _():
                _gstart(j0 + 1, rows1, gsem1)

            _scale(rows0, j0)
            pltpu.sync_copy(rows0, acc.at[src_v.at[j0]], add=True)

            @pl.when(j0 + 1 < GCH)
            def _():
                _gwait(j0 + 1, rows1, gsem1)

                @pl.when(j0 + 2 < GCH)
                def _():
                    _gstart(j0 + 2, rows0, gsem0)

                _scale(rows1, j0 + 1)
                pltpu.sync_copy(rows1, acc.at[src_v.at[j0 + 1]],
                                add=True)

            return carry

        lax.fori_loop(0, (GCH + 1) // 2, _pair, 0)

    plsc.subcore_barrier()

    @pl.when(c == 0)
    def _():
        pltpu.sync_copy(acc.at[pl.ds(rbase, ROWS_PER_TILE)],
                        out_hbm.at[0, pl.ds(rbase, ROWS_PER_TILE)])

    @pl.when(c == 1)
    def _():
        pltpu.sync_copy(acc.at[pl.ds(rbase, ROWS_PER_TILE)],
                        out_hbm.at[1, pl.ds(rbase, ROWS_PER_TILE)])


_spmm = functools.partial(
    pl.kernel,
    mesh=plsc.VectorSubcoreMesh(core_axis_name="c", subcore_axis_name="s"),
    out_type=jax.ShapeDtypeStruct((2, N_PAD, DH), jnp.float32),
    scratch_types=[
        pltpu.VMEM((GCH, CHUNK), jnp.int32),
        pltpu.VMEM((GCH, CHUNK), jnp.int32),
        pltpu.VMEM((GCH, CHUNK), jnp.float32),
        pltpu.VMEM((CHUNK, DH), jnp.float32),
        pltpu.VMEM((CHUNK, DH), jnp.float32),
        pltpu.VMEM_SHARED((N_PAD, DH), jnp.float32),
        pltpu.SemaphoreType.DMA,
        pltpu.SemaphoreType.DMA,
    ],
)(_spmm_body)


def _w2_body(w_ref, o_ref):
    o_ref[...] = jnp.dot(w_ref[...], w_ref[...],
                         preferred_element_type=jnp.float32)


_w2 = pl.pallas_call(
    _w2_body,
    out_shape=jax.ShapeDtypeStruct((D, D), jnp.float32),
)


def _mm_body(agg_ref, u_ref, w2_ref, o_ref):
    a0 = agg_ref[0] * u_ref[...]
    a1 = agg_ref[1] * u_ref[...]
    o_ref[...] = (
        jnp.dot(a0, w2_ref[0], preferred_element_type=jnp.float32)
        + jnp.dot(a1, w2_ref[1], preferred_element_type=jnp.float32))


_mm = pl.pallas_call(
    _mm_body,
    grid=(N_U // RB,),
    in_specs=[
        pl.BlockSpec((2, RB, DH), lambda i: (0, i, 0)),
        pl.BlockSpec((RB, 1), lambda i: (i, 0)),
        pl.BlockSpec((2, DH, D), lambda i: (0, 0, 0)),
    ],
    out_specs=pl.BlockSpec((RB, D), lambda i: (i, 0)),
    out_shape=jax.ShapeDtypeStruct((N_U, D), jnp.float32),
)


def kernel(edge_index, edge_values, user_n_j, item_n_j, user_emb, item_emb,
           mean_weight):
    src = edge_index[0].astype(jnp.int32).reshape(N_TILES, NGROUP, GCH, CHUNK)
    dst0 = (edge_index[1].astype(jnp.int32) * 2).reshape(
        N_TILES, NGROUP, GCH, CHUNK)
    dst1 = dst0 + 1
    ev = edge_values.reshape(N_TILES, NGROUP, GCH, CHUNK)
    item2 = item_emb.reshape(2 * N_I, DH)
    agg = _spmm(dst0, dst1, src, ev, item2)[:, :N_U]
    w2 = _w2(mean_weight)
    return _mm(agg, user_n_j, w2.reshape(2, DH, D))


# speedup vs baseline: 1.0491x; 1.0491x over previous
import functools

import jax
import jax.numpy as jnp
from jax import lax
from jax.experimental import pallas as pl
from jax.experimental.pallas import tpu as pltpu
from jax.experimental.pallas import tpu_sc as plsc

N_U = 10000
N_I = 10000
E_TOT = 160000
D = 256
DH = 128
N_TILES = 16
CHUNK = 80
GCH = 25
NGROUP = 5
EDGES_PER_TILE = CHUNK * GCH * NGROUP
N_PAD = 10240
ROWS_PER_TILE = N_PAD // N_TILES
RB = 1000


def _spmm_body(dst0_hbm, dst1_hbm, src_hbm, ev_hbm, item_hbm, out_hbm,
               dst_v, src_v, ev_v, rows0, rows1, acc, gsem0, gsem1):
    c = lax.axis_index("c")
    s = lax.axis_index("s")
    rbase = s * ROWS_PER_TILE

    zero16 = jnp.zeros((16,), jnp.float32)

    def _zb(i, carry):
        for j in range(DH // 16):
            rows0[i, pl.ds(j * 16, 16)] = zero16
        return carry

    lax.fori_loop(0, CHUNK, _zb, 0)
    for r in range(ROWS_PER_TILE // CHUNK):
        pltpu.sync_copy(rows0, acc.at[pl.ds(rbase + r * CHUNK, CHUNK)])
    plsc.subcore_barrier()

    def _scale(buf, j):
        def _edge16(gg, carry2):
            evv = ev_v[j, pl.ds(gg * 16, 16)]
            for lane in range(16):
                val = evv[lane]
                e = gg * 16 + lane
                for k in range(DH // 16):
                    sl = pl.ds(k * 16, 16)
                    buf[e, sl] = buf[e, sl] * val
            return carry2

        lax.fori_loop(0, CHUNK // 16, _edge16, 0)

    def _gstart(j, buf, sem):
        pltpu.make_async_copy(item_hbm.at[dst_v.at[j]], buf, sem).start()

    def _gwait(j, buf, sem):
        pltpu.make_async_copy(item_hbm.at[dst_v.at[j]], buf, sem).wait()

    for g in range(NGROUP):
        @pl.when(c == 0)
        def _():
            pltpu.sync_copy(dst0_hbm.at[s, g], dst_v)

        @pl.when(c == 1)
        def _():
            pltpu.sync_copy(dst1_hbm.at[s, g], dst_v)

        pltpu.sync_copy(src_hbm.at[s, g], src_v)
        pltpu.sync_copy(ev_hbm.at[s, g], ev_v)

        _gstart(0, rows0, gsem0)

        def _pair(i, carry):
            j0 = i * 2
            _gwait(j0, rows0, gsem0)

            @pl.when(j0 + 1 < GCH)
            def _():
                _gstart(j0 + 1, rows1, gsem1)

            _scale(rows0, j0)
            pltpu.sync_copy(rows0, acc.at[src_v.at[j0]], add=True)

            @pl.when(j0 + 1 < GCH)
            def _():
                _gwait(j0 + 1, rows1, gsem1)

                @pl.when(j0 + 2 < GCH)
                def _():
                    _gstart(j0 + 2, rows0, gsem0)

                _scale(rows1, j0 + 1)
                pltpu.sync_copy(rows1, acc.at[src_v.at[j0 + 1]],
                                add=True)

            return carry

        lax.fori_loop(0, (GCH + 1) // 2, _pair, 0)

    plsc.subcore_barrier()

    @pl.when(c == 0)
    def _():
        pltpu.sync_copy(acc.at[pl.ds(rbase, ROWS_PER_TILE)],
                        out_hbm.at[0, pl.ds(rbase, ROWS_PER_TILE)])

    @pl.when(c == 1)
    def _():
        pltpu.sync_copy(acc.at[pl.ds(rbase, ROWS_PER_TILE)],
                        out_hbm.at[1, pl.ds(rbase, ROWS_PER_TILE)])


_spmm = functools.partial(
    pl.kernel,
    mesh=plsc.VectorSubcoreMesh(core_axis_name="c", subcore_axis_name="s"),
    out_type=jax.ShapeDtypeStruct((2, N_PAD, DH), jnp.float32),
    scratch_types=[
        pltpu.VMEM((GCH, CHUNK), jnp.int32),
        pltpu.VMEM((GCH, CHUNK), jnp.int32),
        pltpu.VMEM((GCH, CHUNK), jnp.float32),
        pltpu.VMEM((CHUNK, DH), jnp.float32),
        pltpu.VMEM((CHUNK, DH), jnp.float32),
        pltpu.VMEM_SHARED((N_PAD, DH), jnp.float32),
        pltpu.SemaphoreType.DMA,
        pltpu.SemaphoreType.DMA,
    ],
)(_spmm_body)


def _w2_body(w_ref, o_ref):
    o_ref[...] = jnp.dot(w_ref[...], w_ref[...],
                         preferred_element_type=jnp.float32)


_w2 = pl.pallas_call(
    _w2_body,
    out_shape=jax.ShapeDtypeStruct((D, D), jnp.float32),
)


def _mm_body(agg_ref, u_ref, w2_ref, o_ref):
    a0 = agg_ref[0] * u_ref[...]
    a1 = agg_ref[1] * u_ref[...]
    o_ref[...] = (
        jnp.dot(a0, w2_ref[0], preferred_element_type=jnp.float32)
        + jnp.dot(a1, w2_ref[1], preferred_element_type=jnp.float32))


_mm = pl.pallas_call(
    _mm_body,
    grid=(N_U // RB,),
    in_specs=[
        pl.BlockSpec((2, RB, DH), lambda i: (0, i, 0)),
        pl.BlockSpec((RB, 1), lambda i: (i, 0)),
        pl.BlockSpec((2, DH, D), lambda i: (0, 0, 0)),
    ],
    out_specs=pl.BlockSpec((RB, D), lambda i: (i, 0)),
    out_shape=jax.ShapeDtypeStruct((N_U, D), jnp.float32),
)


def kernel(edge_index, edge_values, user_n_j, item_n_j, user_emb, item_emb,
           mean_weight):
    src = edge_index[0].astype(jnp.int32).reshape(N_TILES, NGROUP, GCH, CHUNK)
    dst0 = (edge_index[1].astype(jnp.int32) * 2).reshape(
        N_TILES, NGROUP, GCH, CHUNK)
    dst1 = dst0 + 1
    ev = edge_values.reshape(N_TILES, NGROUP, GCH, CHUNK)
    item2 = item_emb.reshape(2 * N_I, DH)
    agg = _spmm(dst0, dst1, src, ev, item2)[:, :N_U]
    w2 = _w2(mean_weight)
    return _mm(agg, user_n_j, w2.reshape(2, DH, D))


# speedup vs baseline: 1.2061x; 1.1496x over previous
import functools

import jax
import jax.numpy as jnp
from jax import lax
from jax.experimental import pallas as pl
from jax.experimental.pallas import tpu as pltpu
from jax.experimental.pallas import tpu_sc as plsc

N_U = 10000
N_I = 10000
E_TOT = 160000
D = 256
DH = 128
N_TILES = 16
CHUNK = 80
GCH = 25
NGROUP = 5
EDGES_PER_TILE = CHUNK * GCH * NGROUP
N_PAD = 10240
ROWS_PER_TILE = N_PAD // N_TILES
RB = 1000


def _spmm_body(dst0_hbm, dst1_hbm, src_hbm, ev_hbm, item_hbm, out_hbm,
               dst_v, src_v, ev_v, rows0, rows1, rows2, acc,
               gsem0, gsem1, gsem2, ssem0, ssem1, ssem2):
    c = lax.axis_index("c")
    s = lax.axis_index("s")
    rbase = s * ROWS_PER_TILE

    zero16 = jnp.zeros((16,), jnp.float32)

    def _zb(i, carry):
        for j in range(DH // 16):
            rows0[i, pl.ds(j * 16, 16)] = zero16
        return carry

    lax.fori_loop(0, CHUNK, _zb, 0)
    for r in range(ROWS_PER_TILE // CHUNK):
        pltpu.sync_copy(rows0, acc.at[pl.ds(rbase + r * CHUNK, CHUNK)])
    plsc.subcore_barrier()

    def _scale(buf, j):
        def _edge16(gg, carry2):
            evv = ev_v[j, pl.ds(gg * 16, 16)]
            for lane in range(16):
                val = evv[lane]
                e = gg * 16 + lane
                for k in range(DH // 16):
                    sl = pl.ds(k * 16, 16)
                    buf[e, sl] = buf[e, sl] * val
            return carry2

        lax.fori_loop(0, CHUNK // 16, _edge16, 0)

    def _gstart(j, buf, sem):
        pltpu.make_async_copy(item_hbm.at[dst_v.at[j]], buf, sem).start()

    def _gwait(j, buf, sem):
        pltpu.make_async_copy(item_hbm.at[dst_v.at[j]], buf, sem).wait()

    def _sstart(j, buf, sem):
        pltpu.async_copy(buf, acc.at[src_v.at[j]], sem, add=True)

    def _swait(j, buf, sem):
        pltpu.make_async_copy(buf, acc.at[src_v.at[j]], sem).wait()

    bufs = ((rows0, gsem0, ssem0), (rows1, gsem1, ssem1),
            (rows2, gsem2, ssem2))

    for g in range(NGROUP):
        @pl.when(c == 0)
        def _():
            pltpu.sync_copy(dst0_hbm.at[s, g], dst_v)

        @pl.when(c == 1)
        def _():
            pltpu.sync_copy(dst1_hbm.at[s, g], dst_v)

        pltpu.sync_copy(src_hbm.at[s, g], src_v)
        pltpu.sync_copy(ev_hbm.at[s, g], ev_v)

        _gstart(0, rows0, gsem0)
        _gstart(1, rows1, gsem1)

        def _tri(i, carry):
            j0 = i * 3
            for k in range(3):
                bufk, gsemk, ssemk = bufs[k]
                bufn, gsemn, ssemn = bufs[(k + 2) % 3]
                j = j0 + k

                @pl.when(j < GCH)
                def _():
                    _gwait(j, bufk, gsemk)

                    @pl.when(j + 2 < GCH)
                    def _():
                        @pl.when(j >= 1)
                        def _():
                            _swait(j - 1, bufn, ssemn)

                        _gstart(j + 2, bufn, gsemn)

                    _scale(bufk, j)
                    _sstart(j, bufk, ssemk)
            return carry

        lax.fori_loop(0, (GCH + 2) // 3, _tri, 0)
        for j in (GCH - 3, GCH - 2, GCH - 1):
            _swait(j, bufs[j % 3][0], bufs[j % 3][2])

    plsc.subcore_barrier()

    @pl.when(c == 0)
    def _():
        pltpu.sync_copy(acc.at[pl.ds(rbase, ROWS_PER_TILE)],
                        out_hbm.at[0, pl.ds(rbase, ROWS_PER_TILE)])

    @pl.when(c == 1)
    def _():
        pltpu.sync_copy(acc.at[pl.ds(rbase, ROWS_PER_TILE)],
                        out_hbm.at[1, pl.ds(rbase, ROWS_PER_TILE)])


_spmm = functools.partial(
    pl.kernel,
    mesh=plsc.VectorSubcoreMesh(core_axis_name="c", subcore_axis_name="s"),
    out_type=jax.ShapeDtypeStruct((2, N_PAD, DH), jnp.float32),
    scratch_types=[
        pltpu.VMEM((GCH, CHUNK), jnp.int32),
        pltpu.VMEM((GCH, CHUNK), jnp.int32),
        pltpu.VMEM((GCH, CHUNK), jnp.float32),
        pltpu.VMEM((CHUNK, DH), jnp.float32),
        pltpu.VMEM((CHUNK, DH), jnp.float32),
        pltpu.VMEM((CHUNK, DH), jnp.float32),
        pltpu.VMEM_SHARED((N_PAD, DH), jnp.float32),
        pltpu.SemaphoreType.DMA,
        pltpu.SemaphoreType.DMA,
        pltpu.SemaphoreType.DMA,
        pltpu.SemaphoreType.DMA,
        pltpu.SemaphoreType.DMA,
        pltpu.SemaphoreType.DMA,
    ],
)(_spmm_body)


def _w2_body(w_ref, o_ref):
    o_ref[...] = jnp.dot(w_ref[...], w_ref[...],
                         preferred_element_type=jnp.float32)


_w2 = pl.pallas_call(
    _w2_body,
    out_shape=jax.ShapeDtypeStruct((D, D), jnp.float32),
)


def _mm_body(agg_ref, u_ref, w2_ref, o_ref):
    a0 = agg_ref[0] * u_ref[...]
    a1 = agg_ref[1] * u_ref[...]
    o_ref[...] = (
        jnp.dot(a0, w2_ref[0], preferred_element_type=jnp.float32)
        + jnp.dot(a1, w2_ref[1], preferred_element_type=jnp.float32))


_mm = pl.pallas_call(
    _mm_body,
    grid=(N_U // RB,),
    in_specs=[
        pl.BlockSpec((2, RB, DH), lambda i: (0, i, 0)),
        pl.BlockSpec((RB, 1), lambda i: (i, 0)),
        pl.BlockSpec((2, DH, D), lambda i: (0, 0, 0)),
    ],
    out_specs=pl.BlockSpec((RB, D), lambda i: (i, 0)),
    out_shape=jax.ShapeDtypeStruct((N_U, D), jnp.float32),
)


def kernel(edge_index, edge_values, user_n_j, item_n_j, user_emb, item_emb,
           mean_weight):
    src = edge_index[0].astype(jnp.int32).reshape(N_TILES, NGROUP, GCH, CHUNK)
    dst0 = (edge_index[1].astype(jnp.int32) * 2).reshape(
        N_TILES, NGROUP, GCH, CHUNK)
    dst1 = dst0 + 1
    ev = edge_values.reshape(N_TILES, NGROUP, GCH, CHUNK)
    item2 = item_emb.reshape(2 * N_I, DH)
    agg = _spmm(dst0, dst1, src, ev, item2)[:, :N_U]
    w2 = _w2(mean_weight)
    return _mm(agg, user_n_j, w2.reshape(2, DH, D))
